# Initial kernel scaffold; baseline (speedup 1.0000x reference)
import jax
import jax.numpy as jnp
from jax.experimental import pallas as pl


def kernel(input_ids, labels, W):
    raise NotImplementedError("write your pallas kernel here")


# speedup vs baseline: 1.0459x; 1.0459x over previous
import functools

import jax
import jax.numpy as jnp
from jax import lax
from jax.experimental import pallas as pl
from jax.experimental.pallas import tpu as pltpu
from jax.experimental.pallas import tpu_sc as plsc

_VOCAB = 100000
_D = 1024
_BATCH = 4
_SEQ = 2048
_ROWS = _BATCH * _SEQ
_NC = 2
_NS = 16
_NW = _NC * _NS
_ROWS_PER_W = _ROWS // _NW
_CHUNK = 32
_NCHUNK = _ROWS_PER_W // _CHUNK


@functools.partial(
    pl.kernel,
    mesh=plsc.VectorSubcoreMesh(core_axis_name="c", subcore_axis_name="s"),
    out_type=jax.ShapeDtypeStruct((_ROWS, _D), jnp.float32),
    scratch_types=[
        pltpu.VMEM((_NCHUNK, _CHUNK), jnp.int32),
        pltpu.VMEM((_CHUNK, _D), jnp.float32),
        pltpu.VMEM((_CHUNK, _D), jnp.float32),
        pltpu.SemaphoreType.DMA,
        pltpu.SemaphoreType.DMA,
        pltpu.SemaphoreType.DMA,
        pltpu.SemaphoreType.DMA,
    ],
)
def _gather_kernel(ids_hbm, table_hbm, out_hbm, idx_v, buf0, buf1,
                   gs0, gs1, ws0, ws1):
    wid = lax.axis_index("s") * _NC + lax.axis_index("c")
    base = wid * _ROWS_PER_W

    pltpu.sync_copy(ids_hbm.at[wid], idx_v)

    bufs = (buf0, buf1)
    gsems = (gs0, gs1)
    wsems = (ws0, ws1)
    gcopy = [None, None]
    wcopy = [None, None]

    gcopy[0] = pltpu.async_copy(table_hbm.at[idx_v.at[0]], bufs[0], gsems[0])
    for c in range(_NCHUNK):
        i = c & 1
        ni = i ^ 1
        if c + 1 < _NCHUNK:
            if wcopy[ni] is not None:
                wcopy[ni].wait()
            gcopy[ni] = pltpu.async_copy(
                table_hbm.at[idx_v.at[c + 1]], bufs[ni], gsems[ni])
        gcopy[i].wait()
        wcopy[i] = pltpu.async_copy(
            bufs[i], out_hbm.at[pl.ds(base + c * _CHUNK, _CHUNK)], wsems[i])
    wcopy[0].wait()
    wcopy[1].wait()


def kernel(input_ids, labels, W):
    ids = jnp.transpose(input_ids).reshape(_NW, _NCHUNK, _CHUNK)
    out = _gather_kernel(ids, W)
    return out.reshape(_SEQ, _BATCH, _D), labels


# speedup vs baseline: 1.0595x; 1.0131x over previous
import functools

import jax
import jax.numpy as jnp
from jax import lax
from jax.experimental import pallas as pl
from jax.experimental.pallas import tpu as pltpu
from jax.experimental.pallas import tpu_sc as plsc

_VOCAB = 100000
_D = 1024
_BATCH = 4
_SEQ = 2048
_ROWS = _BATCH * _SEQ
_NC = 2
_NS = 16
_NW = _NC * _NS
_ROWS_PER_W = _ROWS // _NW
_CHUNK = 32
_NCHUNK = _ROWS_PER_W // _CHUNK
_NBUF = 3


@functools.partial(
    pl.kernel,
    mesh=plsc.VectorSubcoreMesh(core_axis_name="c", subcore_axis_name="s"),
    out_type=jax.ShapeDtypeStruct((_ROWS, _D), jnp.float32),
    scratch_types=(
        [pltpu.VMEM((_NCHUNK, _CHUNK), jnp.int32)]
        + [pltpu.VMEM((_CHUNK, _D), jnp.float32)] * _NBUF
        + [pltpu.SemaphoreType.DMA] * (2 * _NBUF)
    ),
)
def _gather_kernel(ids_hbm, table_hbm, out_hbm, idx_v, *rest):
    bufs = rest[:_NBUF]
    gsems = rest[_NBUF:2 * _NBUF]
    wsems = rest[2 * _NBUF:]

    wid = lax.axis_index("s") * _NC + lax.axis_index("c")
    base = wid * _ROWS_PER_W

    pltpu.sync_copy(ids_hbm.at[wid], idx_v)

    def start_gather(c):
        return pltpu.async_copy(
            table_hbm.at[idx_v.at[c]], bufs[c % _NBUF], gsems[c % _NBUF])

    def start_write(c):
        return pltpu.async_copy(
            bufs[c % _NBUF],
            out_hbm.at[pl.ds(base + c * _CHUNK, _CHUNK)],
            wsems[c % _NBUF])

    gcopy = [None] * _NBUF
    wcopy = [None] * _NBUF
    for c in range(min(_NBUF - 1, _NCHUNK)):
        gcopy[c % _NBUF] = start_gather(c)
    for c in range(_NCHUNK):
        i = c % _NBUF
        nxt = c + _NBUF - 1
        if nxt < _NCHUNK:
            j = nxt % _NBUF
            if wcopy[j] is not None:
                wcopy[j].wait()
            gcopy[j] = start_gather(nxt)
        gcopy[i].wait()
        wcopy[i] = start_write(c)
    for w in wcopy:
        if w is not None:
            w.wait()


def kernel(input_ids, labels, W):
    ids = jnp.transpose(input_ids).reshape(_NW, _NCHUNK, _CHUNK)
    out = _gather_kernel(ids, W)
    return out.reshape(_SEQ, _BATCH, _D), labels
